# baseline (speedup 1.0000x reference)
import functools

import jax
import jax.numpy as jnp
from jax import lax
from jax.experimental import pallas as pl
from jax.experimental.pallas import tpu as pltpu
from jax.experimental.pallas import tpu_sc as plsc

MINPOS = -32
NBIN = 65
B = 2
L = 512
D = 128
NC = 2
NS = 16
NW = NC * NS
PAIRS = B * L
PAIRS_PER_W = PAIRS // NW
CHUNK = 128
NCHUNK = L // CHUNK


def _sc_body(
    x_hbm, idx_hbm, tab_hbm, out_hbm,
    idx_v, ib_v, xbuf, tab_v, sem_in, sem_out,
):
    cid = lax.axis_index("c")
    sid = lax.axis_index("s")
    wid = sid * NC + cid

    pltpu.sync_copy(idx_hbm, idx_v)
    pltpu.sync_copy(tab_hbm, tab_v)

    nchunks = PAIRS_PER_W * NCHUNK

    def chunk_row0(t):
        pair = wid * PAIRS_PER_W + (t // NCHUNK)
        return pair, pair * L + (t % NCHUNK) * CHUNK

    def issue_in(t, k):
        pair, row0 = chunk_row0(t)
        b = pair // L
        jbase = b * L + (t % NCHUNK) * CHUNK
        vi = plsc.load_gather(idx_v, [jnp.full((16,), pair, jnp.int32)])
        for g in range(CHUNK // 16):
            jv = idx_v[pl.ds(jbase + g * 16, 16)]
            ib_v[k, pl.ds(g * 16, 16)] = jnp.clip(jv - vi - MINPOS, 0, NBIN - 1)
        pltpu.async_copy(x_hbm.at[pl.ds(row0, CHUNK)], xbuf.at[k], sem_in.at[k])

    issue_in(0, 0)

    def chunk_body(it, _):
        for k in (0, 1):
            t = 2 * it + k

            @pl.when(t >= 1)
            def _():
                _, prow0 = chunk_row0(t - 1)
                pltpu.make_async_copy(
                    xbuf.at[k ^ 1], out_hbm.at[pl.ds(prow0, CHUNK)],
                    sem_out.at[k ^ 1],
                ).wait()

            @pl.when(t + 1 < nchunks)
            def _():
                issue_in(t + 1, k ^ 1)

            _, row0 = chunk_row0(t)
            pltpu.make_async_copy(
                x_hbm.at[pl.ds(row0, CHUNK)], xbuf.at[k], sem_in.at[k]
            ).wait()

            def add_row(r, _c):
                base = ib_v[k, pl.ds(r, 16)][0] * D
                for c in range(D // 16):
                    sl = pl.ds(c * 16, 16)
                    xbuf[k, r, sl] = xbuf[k, r, sl] + tab_v[pl.ds(base + c * 16, 16)]
                return _c

            lax.fori_loop(0, CHUNK, add_row, 0)

            pltpu.async_copy(
                xbuf.at[k], out_hbm.at[pl.ds(row0, CHUNK)], sem_out.at[k]
            )
        return _

    lax.fori_loop(0, nchunks // 2, chunk_body, 0)

    t = nchunks - 1
    k = t % 2
    _, row0 = chunk_row0(t)
    pltpu.make_async_copy(
        xbuf.at[k], out_hbm.at[pl.ds(row0, CHUNK)], sem_out.at[k]
    ).wait()


def kernel(x, idx, emb_table):
    idx32 = idx.astype(jnp.int32).reshape(B * L)
    x_flat = x.reshape(B * L * L, D)
    tab_flat = emb_table.reshape(NBIN * D)
    mesh = plsc.VectorSubcoreMesh(core_axis_name="c", subcore_axis_name="s")
    out = pl.kernel(
        _sc_body,
        out_type=jax.ShapeDtypeStruct((B * L * L, D), jnp.float32),
        mesh=mesh,
        compiler_params=pltpu.CompilerParams(needs_layout_passes=False),
        scratch_types=[
            pltpu.VMEM((B * L,), jnp.int32),
            pltpu.VMEM((2, CHUNK + 16), jnp.int32),
            pltpu.VMEM((2, CHUNK, D), jnp.float32),
            pltpu.VMEM((NBIN * D,), jnp.float32),
            pltpu.SemaphoreType.DMA((2,)),
            pltpu.SemaphoreType.DMA((2,)),
        ],
    )(x_flat, idx32, tab_flat)
    return out.reshape(B, L, L, D)


# speedup vs baseline: 3.7301x; 3.7301x over previous
import functools

import jax
import jax.numpy as jnp
from jax import lax
from jax.experimental import pallas as pl
from jax.experimental.pallas import tpu as pltpu
from jax.experimental.pallas import tpu_sc as plsc

MINPOS = -32
NBIN = 65
B = 2
L = 512
D = 128
NC = 2
NS = 16
NW = NC * NS
PAIRS = B * L
PAIRS_PER_W = PAIRS // NW
CHUNK = 128
NCHUNK = L // CHUNK
GSEG = 128


def _sc_body(
    x_hbm, idx_hbm, tab_hbm, out_hbm,
    idx_v, ib_v, xbuf, sem_in, tab_sp,
):
    cid = lax.axis_index("c")
    sid = lax.axis_index("s")
    wid = sid * NC + cid

    @pl.when(sid == 0)
    def _():
        pltpu.sync_copy(tab_hbm, tab_sp)

    pltpu.sync_copy(idx_hbm, idx_v)
    plsc.subcore_barrier()

    nchunks = PAIRS_PER_W * NCHUNK

    def chunk_row0(t):
        pair = wid * PAIRS_PER_W + (t // NCHUNK)
        return pair, pair * L + (t % NCHUNK) * CHUNK

    def stage_in(t, k):
        pair, row0 = chunk_row0(t)
        b = pair // L
        jbase = b * L + (t % NCHUNK) * CHUNK
        vi = plsc.load_gather(idx_v, [jnp.full((16,), pair, jnp.int32)])
        for g in range(CHUNK // 16):
            jv = idx_v[pl.ds(jbase + g * 16, 16)]
            ib_v[k, pl.ds(g * 16, 16)] = jnp.clip(jv - vi - MINPOS, 0, NBIN - 1)
        pltpu.async_copy(x_hbm.at[pl.ds(row0, CHUNK)], xbuf.at[k], sem_in.at[k])

    stage_in(0, 0)

    def chunk_body(it, _):
        for k in (0, 1):
            t = 2 * it + k

            @pl.when(t + 1 < nchunks)
            def _():
                stage_in(t + 1, k ^ 1)

            _, row0 = chunk_row0(t)
            pltpu.make_async_copy(
                x_hbm.at[pl.ds(row0, CHUNK)], xbuf.at[k], sem_in.at[k]
            ).wait()
            for s in range(CHUNK // GSEG):
                pltpu.sync_copy(
                    tab_sp.at[ib_v.at[k, pl.ds(s * GSEG, GSEG)]],
                    xbuf.at[k, pl.ds(s * GSEG, GSEG)],
                    add=True,
                )
            pltpu.sync_copy(xbuf.at[k], out_hbm.at[pl.ds(row0, CHUNK)])
        return _

    lax.fori_loop(0, nchunks // 2, chunk_body, 0)


def kernel(x, idx, emb_table):
    idx32 = idx.astype(jnp.int32).reshape(B * L)
    x_flat = x.reshape(B * L * L, D)
    mesh = plsc.VectorSubcoreMesh(core_axis_name="c", subcore_axis_name="s")
    out = pl.kernel(
        _sc_body,
        out_type=jax.ShapeDtypeStruct((B * L * L, D), jnp.float32),
        mesh=mesh,
        compiler_params=pltpu.CompilerParams(needs_layout_passes=False),
        scratch_types=[
            pltpu.VMEM((B * L,), jnp.int32),
            pltpu.VMEM((2, CHUNK), jnp.int32),
            pltpu.VMEM((2, CHUNK, D), jnp.float32),
            pltpu.SemaphoreType.DMA((2,)),
            pltpu.VMEM_SHARED((NBIN, D), jnp.float32),
        ],
    )(x_flat, idx32, emb_table)
    return out.reshape(B, L, L, D)
